# Initial kernel scaffold; baseline (speedup 1.0000x reference)
import jax
import jax.numpy as jnp
from jax.experimental import pallas as pl


def kernel(x, edge_index, edge_weight, W1, b1, W2, b2):
    raise NotImplementedError("write your pallas kernel here")


# speedup vs baseline: 11.0496x; 11.0496x over previous
import functools

import jax
import jax.numpy as jnp
from jax import lax
from jax.experimental import pallas as pl
import jax.experimental.pallas.tpu as pltpu
from jax.experimental.pallas import tpu_sc as plsc

N = 10000
NP = 10240
D = 128
E = 320000
CH = 128
NCHUNK = 79
NC = 2
NS = 16
NW = NC * NS
EPT = NCHUNK * CH
EP = NW * EPT
NPT = NP // NS

_MESH = plsc.VectorSubcoreMesh(core_axis_name="c", subcore_axis_name="s",
                               num_cores=NC, num_subcores=NS)
_SC_PARAMS = pltpu.CompilerParams(needs_layout_passes=False)


def _wid(c, s):
    return s * NC + c


@functools.partial(
    pl.kernel,
    out_type=jax.ShapeDtypeStruct((NC, NP), jnp.float32),
    mesh=_MESH,
    compiler_params=_SC_PARAMS,
    scratch_types=[
        pltpu.VMEM((NCHUNK, CH), jnp.int32),
        pltpu.VMEM((EPT,), jnp.float32),
        pltpu.VMEM((NPT,), jnp.float32),
        pltpu.VMEM_SHARED((NP,), jnp.float32),
    ],
)
def _deg_kernel(dst_hbm, ew_hbm, out_hbm, dst_v, ew_v, stage_v, deg_sh):
    c = lax.axis_index("c")
    s = lax.axis_index("s")
    w = _wid(c, s)
    pltpu.sync_copy(dst_hbm.at[w], dst_v)
    pltpu.sync_copy(ew_hbm.at[pl.ds(w * EPT, EPT)], ew_v)

    def zero_body(i, _):
        stage_v[pl.ds(i * 16, 16)] = jnp.zeros((16,), jnp.float32)
        return 0
    lax.fori_loop(0, NPT // 16, zero_body, 0)
    pltpu.sync_copy(stage_v, deg_sh.at[pl.ds(s * NPT, NPT)])
    plsc.subcore_barrier()

    def chunk_body(i, _):
        pltpu.sync_copy(ew_v.at[pl.ds(i * CH, CH)],
                        deg_sh.at[dst_v.at[i]], add=True)
        return 0
    lax.fori_loop(0, NCHUNK, chunk_body, 0)
    plsc.subcore_barrier()

    pltpu.sync_copy(deg_sh.at[pl.ds(s * NPT, NPT)], stage_v)
    pltpu.sync_copy(stage_v, out_hbm.at[c, pl.ds(s * NPT, NPT)])


@functools.partial(
    pl.kernel,
    out_type=jax.ShapeDtypeStruct((EP,), jnp.float32),
    mesh=_MESH,
    compiler_params=_SC_PARAMS,
    scratch_types=[
        pltpu.VMEM((NP,), jnp.float32),
        pltpu.VMEM((NCHUNK, CH), jnp.int32),
        pltpu.VMEM((NCHUNK, CH), jnp.int32),
        pltpu.VMEM((EPT,), jnp.float32),
        pltpu.VMEM((EPT,), jnp.float32),
    ],
)
def _norm_kernel(dis_hbm, src_hbm, dst_hbm, ew_hbm, norm_hbm,
                 dis_v, src_v, dst_v, ew_v, norm_v):
    c = lax.axis_index("c")
    s = lax.axis_index("s")
    w = _wid(c, s)
    pltpu.sync_copy(dis_hbm, dis_v)
    pltpu.sync_copy(src_hbm.at[w], src_v)
    pltpu.sync_copy(dst_hbm.at[w], dst_v)
    pltpu.sync_copy(ew_hbm.at[pl.ds(w * EPT, EPT)], ew_v)

    def row_body(r, _):
        for j in range(CH // 16):
            sl = pl.ds(j * 16, 16)
            fl = pl.ds(r * CH + j * 16, 16)
            gs = plsc.load_gather(dis_v, [src_v[r, sl]])
            gd = plsc.load_gather(dis_v, [dst_v[r, sl]])
            norm_v[fl] = gs * ew_v[fl] * gd
        return 0
    lax.fori_loop(0, NCHUNK, row_body, 0)
    pltpu.sync_copy(norm_v, norm_hbm.at[pl.ds(w * EPT, EPT)])


@functools.partial(
    pl.kernel,
    out_type=jax.ShapeDtypeStruct((NC, NP, D), jnp.float32),
    mesh=_MESH,
    compiler_params=_SC_PARAMS,
    scratch_types=[
        pltpu.VMEM((NCHUNK, CH), jnp.int32),
        pltpu.VMEM((NCHUNK, CH), jnp.int32),
        pltpu.VMEM((EPT,), jnp.float32),
        pltpu.VMEM((CH, D), jnp.float32),
        pltpu.VMEM_SHARED((NP, D), jnp.float32),
        pltpu.SemaphoreType.DMA,
    ],
)
def _layer_kernel(h_hbm, src_hbm, dst_hbm, norm_hbm, out_hbm,
                  src_v, dst_v, norm_v, rows_v, acc_sh, sem):
    c = lax.axis_index("c")
    s = lax.axis_index("s")
    w = _wid(c, s)
    pltpu.sync_copy(src_hbm.at[w], src_v)
    pltpu.sync_copy(dst_hbm.at[w], dst_v)
    pltpu.sync_copy(norm_hbm.at[pl.ds(w * EPT, EPT)], norm_v)

    def zero_body(r, _):
        for j in range(D // 16):
            rows_v[r, pl.ds(j * 16, 16)] = jnp.zeros((16,), jnp.float32)
        return 0
    lax.fori_loop(0, CH, zero_body, 0)
    for k in range(NPT // CH):
        pltpu.sync_copy(rows_v, acc_sh.at[pl.ds(s * NPT + k * CH, CH)])
    plsc.subcore_barrier()

    def chunk_body(i, _):
        pltpu.async_copy(h_hbm.at[src_v.at[i]], rows_v, sem).wait()

        def scale_body(g, _):
            nv = norm_v[pl.ds(i * CH + g * 16, 16)]
            for t in range(16):
                n = nv[t]
                r = g * 16 + t
                for j in range(D // 16):
                    sl = pl.ds(j * 16, 16)
                    rows_v[r, sl] = rows_v[r, sl] * n
            return 0
        lax.fori_loop(0, CH // 16, scale_body, 0)
        pltpu.sync_copy(rows_v, acc_sh.at[dst_v.at[i]], add=True)
        return 0
    lax.fori_loop(0, NCHUNK, chunk_body, 0)
    plsc.subcore_barrier()

    for k in range(NPT // CH):
        sl = pl.ds(s * NPT + k * CH, CH)
        pltpu.sync_copy(acc_sh.at[sl], rows_v)
        pltpu.sync_copy(rows_v, out_hbm.at[c, sl])


def _prep_body(degp_ref, dis_ref, selfw_ref):
    p = degp_ref[...]
    deg = p[: NP // D] + p[NP // D:] + 1.0
    dis_ref[...] = lax.rsqrt(deg)
    selfw_ref[...] = 1.0 / deg


def _matmul_body(x_ref, w_ref, o_ref):
    o_ref[...] = jnp.dot(x_ref[...], w_ref[...],
                         preferred_element_type=jnp.float32)


def _mid_body(p0_ref, p1_ref, h_ref, sw_ref, b_ref, w_ref, o_ref):
    y = p0_ref[...] + p1_ref[...] + h_ref[...] * sw_ref[...] + b_ref[...]
    y = jnp.maximum(y, 0.0)
    o_ref[...] = jnp.dot(y, w_ref[...], preferred_element_type=jnp.float32)


def _final_body(p0_ref, p1_ref, h_ref, sw_ref, b_ref, o_ref):
    o_ref[...] = (p0_ref[...] + p1_ref[...] + h_ref[...] * sw_ref[...]
                  + b_ref[...])


_MB = 1024
_GRID = (NP // _MB,)


def _blk(shape, imap):
    return pl.BlockSpec(shape, imap)


def _tc_matmul(x, w):
    return pl.pallas_call(
        _matmul_body,
        grid=_GRID,
        in_specs=[_blk((_MB, D), lambda i: (i, 0)),
                  _blk((D, D), lambda i: (0, 0))],
        out_specs=_blk((_MB, D), lambda i: (i, 0)),
        out_shape=jax.ShapeDtypeStruct((NP, D), jnp.float32),
    )(x, w)


def _tc_prep(deg_partials):
    return pl.pallas_call(
        _prep_body,
        in_specs=[_blk((2 * NP // D, D), lambda: (0, 0))],
        out_specs=[_blk((NP // D, D), lambda: (0, 0))] * 2,
        out_shape=[jax.ShapeDtypeStruct((NP // D, D), jnp.float32)] * 2,
    )(deg_partials)


def _tc_mid(p0, p1, h, sw, b, w):
    return pl.pallas_call(
        _mid_body,
        grid=_GRID,
        in_specs=[_blk((_MB, D), lambda i: (i, 0)),
                  _blk((_MB, D), lambda i: (i, 0)),
                  _blk((_MB, D), lambda i: (i, 0)),
                  _blk((_MB, 1), lambda i: (i, 0)),
                  _blk((1, D), lambda i: (0, 0)),
                  _blk((D, D), lambda i: (0, 0))],
        out_specs=_blk((_MB, D), lambda i: (i, 0)),
        out_shape=jax.ShapeDtypeStruct((NP, D), jnp.float32),
    )(p0, p1, h, sw, b, w)


def _tc_final(p0, p1, h, sw, b):
    return pl.pallas_call(
        _final_body,
        grid=_GRID,
        in_specs=[_blk((_MB, D), lambda i: (i, 0)),
                  _blk((_MB, D), lambda i: (i, 0)),
                  _blk((_MB, D), lambda i: (i, 0)),
                  _blk((_MB, 1), lambda i: (i, 0)),
                  _blk((1, D), lambda i: (0, 0))],
        out_specs=_blk((_MB, D), lambda i: (i, 0)),
        out_shape=jax.ShapeDtypeStruct((NP, D), jnp.float32),
    )(p0, p1, h, sw, b)


def kernel(x, edge_index, edge_weight, W1, b1, W2, b2):
    src = edge_index[0].astype(jnp.int32)
    dst = edge_index[1].astype(jnp.int32)
    ew = edge_weight.astype(jnp.float32)

    pad_e = EP - E
    src_p = jnp.concatenate([src, jnp.zeros((pad_e,), jnp.int32)])
    dst_p = jnp.concatenate([dst, jnp.zeros((pad_e,), jnp.int32)])
    ew_p = jnp.concatenate([ew, jnp.zeros((pad_e,), jnp.float32)])
    src3d = src_p.reshape(NW, NCHUNK, CH)
    dst3d = dst_p.reshape(NW, NCHUNK, CH)
    x_p = jnp.concatenate(
        [x.astype(jnp.float32), jnp.zeros((NP - N, D), jnp.float32)])

    deg_partials = _deg_kernel(dst3d, ew_p)
    dis2d, selfw2d = _tc_prep(deg_partials.reshape(2 * NP // D, D))
    dis = dis2d.reshape(NP)
    sw = selfw2d.reshape(NP, 1)

    norm = _norm_kernel(dis, src3d, dst3d, ew_p)

    h1 = _tc_matmul(x_p, W1)
    p1 = _layer_kernel(h1, src3d, dst3d, norm)
    h2 = _tc_mid(p1[0], p1[1], h1, sw, b1.reshape(1, D), W2)
    p2 = _layer_kernel(h2, src3d, dst3d, norm)
    out = _tc_final(p2[0], p2[1], h2, sw, b2.reshape(1, D))
    return out[:N]
